# baseline (speedup 1.0000x reference)
import jax
import jax.numpy as jnp
from jax import lax
from jax.experimental import pallas as pl
from jax.experimental.pallas import tpu as pltpu
from jax.experimental.pallas import tpu_sc as plsc
import functools

TOTAL = 32768
B = 16
D = 128
LANES = 16
NC = 2
NS = 16
NW = NC * NS
ROWS_PER_W = TOTAL // NW
CHUNK = 256
NCHUNK = ROWS_PER_W // CHUNK
DV = D // LANES


def _sc_partial_sums(h_hbm, ptr_hbm, out_hbm, ptr_v, buf0, buf1, acc,
                     sem0, sem1):
  wid = lax.axis_index("s") * NC + lax.axis_index("c")
  base = wid * ROWS_PER_W
  bufs = (buf0, buf1)
  sems = (sem0, sem1)

  pending = {}
  for c in range(min(2, NCHUNK)):
    pending[c] = pltpu.async_copy(
        h_hbm.at[pl.ds(base + c * CHUNK, CHUNK)], bufs[c % 2], sems[c % 2])

  pltpu.sync_copy(ptr_hbm.at[pl.ds(0, LANES)], ptr_v)
  ptr_vec = ptr_v[...]
  ptrs = [jnp.int32(0)]
  for b in range(1, B):
    ptrs.append(ptr_vec[b])
  ptrs.append(jnp.int32(TOTAL))

  zero = jnp.zeros((LANES,), jnp.float32)
  for b in range(B):
    for j in range(DV):
      acc[b, pl.ds(j * LANES, LANES)] = zero

  for c in range(NCHUNK):
    clo = base + c * CHUNK
    buf = bufs[c % 2]
    pending[c].wait()
    for b in range(B):
      s_loc = jnp.clip(ptrs[b] - clo, 0, CHUNK)
      e_loc = jnp.clip(ptrs[b + 1] - clo, 0, CHUNK)

      @pl.when(e_loc > s_loc)
      def _():
        @plsc.parallel_loop(s_loc, e_loc, step=1, unroll=2,
                            carry=(zero,) * DV)
        def sums(r, carry):
          return tuple(carry[j] + buf[r, pl.ds(j * LANES, LANES)]
                       for j in range(DV))
        for j in range(DV):
          acc[b, pl.ds(j * LANES, LANES)] = (
              acc[b, pl.ds(j * LANES, LANES)] + sums[j])

    if c + 2 < NCHUNK:
      pending[c + 2] = pltpu.async_copy(
          h_hbm.at[pl.ds(base + (c + 2) * CHUNK, CHUNK)], buf, sems[c % 2])

  pltpu.sync_copy(acc, out_hbm.at[wid])


@functools.partial(
    pl.kernel,
    out_type=jax.ShapeDtypeStruct((NW, B, D), jnp.float32),
    mesh=plsc.VectorSubcoreMesh(core_axis_name="c", subcore_axis_name="s"),
    scratch_types=[
        pltpu.VMEM((LANES,), jnp.int32),
        pltpu.VMEM((CHUNK, D), jnp.float32),
        pltpu.VMEM((CHUNK, D), jnp.float32),
        pltpu.VMEM((B, D), jnp.float32),
        pltpu.SemaphoreType.DMA,
        pltpu.SemaphoreType.DMA,
    ],
)
def _partial_sums(h_hbm, ptr_hbm, out_hbm, ptr_v, buf0, buf1, acc,
                  sem0, sem1):
  _sc_partial_sums(h_hbm, ptr_hbm, out_hbm, ptr_v, buf0, buf1, acc,
                   sem0, sem1)


def _combine_body(partial_ref, ptr_ref, out_ref):
  sums = jnp.sum(partial_ref[...], axis=0)
  cnt = jnp.stack([ptr_ref[b + 1] - ptr_ref[b] for b in range(B)])
  denom = jnp.maximum(cnt.astype(jnp.float32), 1.0)[:, None]
  out_ref[...] = sums / denom


def kernel(H, bag_ptr):
  partial = _partial_sums(H, bag_ptr)
  out = pl.pallas_call(
      _combine_body,
      out_shape=jax.ShapeDtypeStruct((B, D), jnp.float32),
      in_specs=[
          pl.BlockSpec(memory_space=pltpu.VMEM),
          pl.BlockSpec(memory_space=pltpu.SMEM),
      ],
      out_specs=pl.BlockSpec(memory_space=pltpu.VMEM),
  )(partial, bag_ptr)
  return out


# speedup vs baseline: 2.8582x; 2.8582x over previous
import jax
import jax.numpy as jnp
from jax import lax
from jax.experimental import pallas as pl
from jax.experimental.pallas import tpu as pltpu
from jax.experimental.pallas import tpu_sc as plsc
import functools

TOTAL = 32768
B = 16
D = 128
LANES = 16
NC = 2
NS = 16
NW = NC * NS
ROWS_PER_W = TOTAL // NW
CHUNK = 256
NCHUNK = ROWS_PER_W // CHUNK
DV = D // LANES


def _sc_partial_sums(h_hbm, ptr_hbm, out_hbm, ptr_v, buf0, buf1, acc,
                     sem0, sem1):
  wid = lax.axis_index("s") * NC + lax.axis_index("c")
  base = wid * ROWS_PER_W
  bufs = (buf0, buf1)
  sems = (sem0, sem1)

  pending = {}
  for c in range(min(2, NCHUNK)):
    pending[c] = pltpu.async_copy(
        h_hbm.at[pl.ds(base + c * CHUNK, CHUNK)], bufs[c % 2], sems[c % 2])

  pltpu.sync_copy(ptr_hbm.at[pl.ds(0, LANES)], ptr_v)
  ptr_vec = ptr_v[...]
  ptrs = [jnp.int32(0)]
  for b in range(1, B):
    ptrs.append(ptr_vec[b])
  ptrs.append(jnp.int32(TOTAL))

  zero = jnp.zeros((LANES,), jnp.float32)
  for b in range(B):
    for j in range(DV):
      acc[b, pl.ds(j * LANES, LANES)] = zero

  for c in range(NCHUNK):
    clo = base + c * CHUNK
    buf = bufs[c % 2]
    pending[c].wait()
    for b in range(B):
      s_loc = jnp.clip(ptrs[b] - clo, 0, CHUNK)
      e_loc = jnp.clip(ptrs[b + 1] - clo, 0, CHUNK)

      @pl.when(e_loc > s_loc)
      def _():
        @plsc.parallel_loop(s_loc, e_loc, step=1, unroll=2,
                            carry=(zero,) * DV)
        def sums(r, carry):
          return tuple(carry[j] + buf[r, pl.ds(j * LANES, LANES)]
                       for j in range(DV))
        for j in range(DV):
          acc[b, pl.ds(j * LANES, LANES)] = (
              acc[b, pl.ds(j * LANES, LANES)] + sums[j])

    if c + 2 < NCHUNK:
      pending[c + 2] = pltpu.async_copy(
          h_hbm.at[pl.ds(base + (c + 2) * CHUNK, CHUNK)], buf, sems[c % 2])

  pltpu.sync_copy(acc, out_hbm.at[wid])


@functools.partial(
    pl.kernel,
    out_type=jax.ShapeDtypeStruct((NW, B, D), jnp.float32),
    mesh=plsc.VectorSubcoreMesh(core_axis_name="c", subcore_axis_name="s"),
    scratch_types=[
        pltpu.VMEM((LANES,), jnp.int32),
        pltpu.VMEM((CHUNK, D), jnp.float32),
        pltpu.VMEM((CHUNK, D), jnp.float32),
        pltpu.VMEM((B, D), jnp.float32),
        pltpu.SemaphoreType.DMA,
        pltpu.SemaphoreType.DMA,
    ],
)
def _partial_sums(h_hbm, ptr_hbm, out_hbm, ptr_v, buf0, buf1, acc,
                  sem0, sem1):
  _sc_partial_sums(h_hbm, ptr_hbm, out_hbm, ptr_v, buf0, buf1, acc,
                   sem0, sem1)


def _combine_body(partial_ref, ptr_ref, out_ref):
  sums = jnp.sum(partial_ref[...], axis=0)
  cnt = jnp.stack([ptr_ref[b + 1] - ptr_ref[b] for b in range(B)])
  denom = jnp.maximum(cnt.astype(jnp.float32), 1.0)[:, None]
  out_ref[...] = sums / denom


TC_BLK = 2048


def _tc_rowsum_body(ptr_ref, h_ref, out_ref):
  i = pl.program_id(0)
  rows = i * TC_BLK + jax.lax.broadcasted_iota(jnp.int32, (1, TC_BLK), 1)
  lower = jnp.stack([ptr_ref[b] for b in range(B)])[:, None]
  upper = jnp.stack([ptr_ref[b + 1] for b in range(B)])[:, None]
  mask = ((rows >= lower) & (rows < upper)).astype(jnp.float32)
  partial = jax.lax.dot_general(
      mask, h_ref[...], (((1,), (0,)), ((), ())),
      preferred_element_type=jnp.float32)

  @pl.when(i == 0)
  def _():
    out_ref[...] = jnp.zeros_like(out_ref)

  out_ref[...] += partial

  @pl.when(i == pl.num_programs(0) - 1)
  def _():
    cnt = jnp.stack([ptr_ref[b + 1] - ptr_ref[b] for b in range(B)])
    denom = jnp.maximum(cnt.astype(jnp.float32), 1.0)[:, None]
    out_ref[...] = out_ref[...] / denom


def _tc_only(H, bag_ptr):
  return pl.pallas_call(
      _tc_rowsum_body,
      grid=(TOTAL // TC_BLK,),
      in_specs=[
          pl.BlockSpec(memory_space=pltpu.SMEM),
          pl.BlockSpec((TC_BLK, D), lambda i: (i, 0)),
      ],
      out_specs=pl.BlockSpec((B, D), lambda i: (0, 0)),
      out_shape=jax.ShapeDtypeStruct((B, D), jnp.float32),
  )(bag_ptr, H)


def kernel(H, bag_ptr):
  return _tc_only(H, bag_ptr)


def _kernel_sc(H, bag_ptr):
  partial = _partial_sums(H, bag_ptr)
  out = pl.pallas_call(
      _combine_body,
      out_shape=jax.ShapeDtypeStruct((B, D), jnp.float32),
      in_specs=[
          pl.BlockSpec(memory_space=pltpu.VMEM),
          pl.BlockSpec(memory_space=pltpu.SMEM),
      ],
      out_specs=pl.BlockSpec(memory_space=pltpu.VMEM),
  )(partial, bag_ptr)
  return out
